# baseline (device time: 211381 ns/iter reference)
import functools

import jax
import jax.numpy as jnp
from jax import lax
from jax.experimental import pallas as pl
from jax.experimental.pallas import tpu as pltpu

N_DEV = 32
M = 1024
N = 1024
CHUNK = M // N_DEV


def _gelu(y):
    c = 0.7978845608028654
    return 0.5 * y * (1.0 + jnp.tanh(c * (y + 0.044715 * y * y * y)))


def kernel(x, w_mat):
    def body(x_ref, w_ref, out_ref, acc_ref, rs_buf,
             rs_send_sems, rs_recv_sems, ag_send_sems, ag_recv_sems):
        me = lax.axis_index("i")
        left = lax.rem(me - 1 + N_DEV, N_DEV)
        right = lax.rem(me + 1, N_DEV)

        barrier_sem = pltpu.get_barrier_semaphore()
        for nbr in [left, right]:
            pl.semaphore_signal(
                barrier_sem, inc=1,
                device_id=(nbr,), device_id_type=pl.DeviceIdType.MESH,
            )
        pl.semaphore_wait(barrier_sem, 2)

        acc_ref[...] = jnp.dot(
            x_ref[...], w_ref[...], preferred_element_type=jnp.float32
        )

        for s in range(N_DEV - 1):
            send_idx = lax.rem(me - s + N_DEV, N_DEV)
            recv_idx = lax.rem(me - s - 1 + 2 * N_DEV, N_DEV)
            rdma = pltpu.make_async_remote_copy(
                src_ref=acc_ref.at[pl.ds(send_idx * CHUNK, CHUNK), :],
                dst_ref=rs_buf.at[s],
                send_sem=rs_send_sems.at[s],
                recv_sem=rs_recv_sems.at[s],
                device_id=(right,),
                device_id_type=pl.DeviceIdType.MESH,
            )
            rdma.start()
            rdma.wait()
            acc_ref[pl.ds(recv_idx * CHUNK, CHUNK), :] = (
                acc_ref[pl.ds(recv_idx * CHUNK, CHUNK), :] + rs_buf[s]
            )

        own = lax.rem(me + 1, N_DEV)
        out_ref[pl.ds(own * CHUNK, CHUNK), :] = _gelu(
            acc_ref[pl.ds(own * CHUNK, CHUNK), :]
        )

        for t in range(N_DEV - 1):
            chunk_idx = lax.rem(own - t + N_DEV, N_DEV)
            rdma = pltpu.make_async_remote_copy(
                src_ref=out_ref.at[pl.ds(chunk_idx * CHUNK, CHUNK), :],
                dst_ref=out_ref.at[pl.ds(chunk_idx * CHUNK, CHUNK), :],
                send_sem=ag_send_sems.at[t],
                recv_sem=ag_recv_sems.at[t],
                device_id=(right,),
                device_id_type=pl.DeviceIdType.MESH,
            )
            rdma.start()
            rdma.wait()

        @functools.partial(
            pl.run_scoped, second_barrier=pltpu.SemaphoreType.REGULAR
        )
        def _(second_barrier):
            for nbr in [left, right]:
                pl.semaphore_signal(
                    second_barrier, inc=1,
                    device_id=(nbr,), device_id_type=pl.DeviceIdType.MESH,
                )
            pl.semaphore_wait(second_barrier, 2)

    return pl.pallas_call(
        body,
        out_shape=jax.ShapeDtypeStruct((M, N), jnp.float32),
        in_specs=[
            pl.BlockSpec(memory_space=pltpu.VMEM),
            pl.BlockSpec(memory_space=pltpu.VMEM),
        ],
        out_specs=pl.BlockSpec(memory_space=pltpu.VMEM),
        scratch_shapes=[
            pltpu.VMEM((M, N), jnp.float32),
            pltpu.VMEM((N_DEV - 1, CHUNK, N), jnp.float32),
            pltpu.SemaphoreType.DMA((N_DEV - 1,)),
            pltpu.SemaphoreType.DMA((N_DEV - 1,)),
            pltpu.SemaphoreType.DMA((N_DEV - 1,)),
            pltpu.SemaphoreType.DMA((N_DEV - 1,)),
        ],
        compiler_params=pltpu.CompilerParams(collective_id=0),
    )(x, w_mat)


# device time: 124039 ns/iter; 1.7041x vs baseline; 1.7041x over previous
import functools

import jax
import jax.numpy as jnp
from jax import lax
from jax.experimental import pallas as pl
from jax.experimental.pallas import tpu as pltpu

N_DEV = 32
M = 1024
N = 1024
CHUNK = M // N_DEV


def _gelu(y):
    c = 0.7978845608028654
    return 0.5 * y * (1.0 + jnp.tanh(c * (y + 0.044715 * y * y * y)))


def kernel(x, w_mat):
    def body(x_ref, w_ref, out_ref, acc_ref, rs_buf,
             rs_send_sems, rs_recv_sems, ag_send_sems, ag_recv_sems):
        me = lax.axis_index("i")

        barrier_sem = pltpu.get_barrier_semaphore()
        for k in range(1, N_DEV):
            pl.semaphore_signal(
                barrier_sem, inc=1,
                device_id=(lax.rem(me + k, N_DEV),),
                device_id_type=pl.DeviceIdType.MESH,
            )
        pl.semaphore_wait(barrier_sem, N_DEV - 1)

        acc_ref[...] = jnp.dot(
            x_ref[...], w_ref[...], preferred_element_type=jnp.float32
        )

        rs_rdmas = []
        for k in range(1, N_DEV):
            peer = lax.rem(me + k, N_DEV)
            rdma = pltpu.make_async_remote_copy(
                src_ref=acc_ref.at[pl.ds(peer * CHUNK, CHUNK), :],
                dst_ref=rs_buf.at[me],
                send_sem=rs_send_sems.at[k],
                recv_sem=rs_recv_sems.at[k],
                device_id=(peer,),
                device_id_type=pl.DeviceIdType.MESH,
            )
            rdma.start()
            rs_rdmas.append((k, rdma))

        own = acc_ref[pl.ds(me * CHUNK, CHUNK), :]
        for k in range(1, N_DEV):
            src = lax.rem(me + N_DEV - k, N_DEV)
            rdma = pltpu.make_async_remote_copy(
                src_ref=acc_ref.at[pl.ds(0, CHUNK), :],
                dst_ref=rs_buf.at[src],
                send_sem=rs_send_sems.at[k],
                recv_sem=rs_recv_sems.at[k],
                device_id=(src,),
                device_id_type=pl.DeviceIdType.MESH,
            )
            rdma.wait_recv()
            own = own + rs_buf[src]

        out_ref[pl.ds(me * CHUNK, CHUNK), :] = _gelu(own)

        ag_rdmas = []
        for k in range(1, N_DEV):
            peer = lax.rem(me + k, N_DEV)
            rdma = pltpu.make_async_remote_copy(
                src_ref=out_ref.at[pl.ds(me * CHUNK, CHUNK), :],
                dst_ref=out_ref.at[pl.ds(me * CHUNK, CHUNK), :],
                send_sem=ag_send_sems.at[k],
                recv_sem=ag_recv_sems.at[k],
                device_id=(peer,),
                device_id_type=pl.DeviceIdType.MESH,
            )
            rdma.start()
            ag_rdmas.append(rdma)

        for _, rdma in rs_rdmas:
            rdma.wait_send()

        for k in range(1, N_DEV):
            src = lax.rem(me + N_DEV - k, N_DEV)
            rdma = pltpu.make_async_remote_copy(
                src_ref=out_ref.at[pl.ds(src * CHUNK, CHUNK), :],
                dst_ref=out_ref.at[pl.ds(src * CHUNK, CHUNK), :],
                send_sem=ag_send_sems.at[k],
                recv_sem=ag_recv_sems.at[k],
                device_id=(src,),
                device_id_type=pl.DeviceIdType.MESH,
            )
            rdma.wait_recv()
        for rdma in ag_rdmas:
            rdma.wait_send()

        @functools.partial(
            pl.run_scoped, second_barrier=pltpu.SemaphoreType.REGULAR
        )
        def _(second_barrier):
            for k in range(1, N_DEV):
                pl.semaphore_signal(
                    second_barrier, inc=1,
                    device_id=(lax.rem(me + k, N_DEV),),
                    device_id_type=pl.DeviceIdType.MESH,
                )
            pl.semaphore_wait(second_barrier, N_DEV - 1)

    return pl.pallas_call(
        body,
        out_shape=jax.ShapeDtypeStruct((M, N), jnp.float32),
        in_specs=[
            pl.BlockSpec(memory_space=pltpu.VMEM),
            pl.BlockSpec(memory_space=pltpu.VMEM),
        ],
        out_specs=pl.BlockSpec(memory_space=pltpu.VMEM),
        scratch_shapes=[
            pltpu.VMEM((M, N), jnp.float32),
            pltpu.VMEM((N_DEV, CHUNK, N), jnp.float32),
            pltpu.SemaphoreType.DMA((N_DEV,)),
            pltpu.SemaphoreType.DMA((N_DEV,)),
            pltpu.SemaphoreType.DMA((N_DEV,)),
            pltpu.SemaphoreType.DMA((N_DEV,)),
        ],
        compiler_params=pltpu.CompilerParams(collective_id=0),
    )(x, w_mat)


# device time: 67325 ns/iter; 3.1397x vs baseline; 1.8424x over previous
import functools

import jax
import jax.numpy as jnp
from jax import lax
from jax.experimental import pallas as pl
from jax.experimental.pallas import tpu as pltpu

N_DEV = 32
M = 1024
N = 1024
CHUNK = M // N_DEV


def _gelu(y):
    c = 0.7978845608028654
    return 0.5 * y * (1.0 + jnp.tanh(c * (y + 0.044715 * y * y * y)))


def kernel(x, w_mat):
    def body(x_ref, w_ref, out_ref, acc_ref, acc16_ref, rs_buf, ag_buf,
             rs_send_sems, rs_recv_sems, ag_send_sems, ag_recv_sems):
        me = lax.axis_index("i")

        barrier_sem = pltpu.get_barrier_semaphore()
        for k in range(1, N_DEV):
            pl.semaphore_signal(
                barrier_sem, inc=1,
                device_id=(lax.rem(me + k, N_DEV),),
                device_id_type=pl.DeviceIdType.MESH,
            )
        pl.semaphore_wait(barrier_sem, N_DEV - 1)

        acc_ref[...] = jnp.dot(
            x_ref[...], w_ref[...], preferred_element_type=jnp.float32
        )
        acc16_ref[...] = acc_ref[...].astype(jnp.bfloat16)

        rs_rdmas = []
        for k in range(1, N_DEV):
            peer = lax.rem(me + k, N_DEV)
            rdma = pltpu.make_async_remote_copy(
                src_ref=acc16_ref.at[pl.ds(peer * CHUNK, CHUNK), :],
                dst_ref=rs_buf.at[me],
                send_sem=rs_send_sems.at[k],
                recv_sem=rs_recv_sems.at[k],
                device_id=(peer,),
                device_id_type=pl.DeviceIdType.MESH,
            )
            rdma.start()
            rs_rdmas.append(rdma)

        own = acc_ref[pl.ds(me * CHUNK, CHUNK), :]
        for k in range(1, N_DEV):
            src = lax.rem(me + N_DEV - k, N_DEV)
            rdma = pltpu.make_async_remote_copy(
                src_ref=acc16_ref.at[pl.ds(0, CHUNK), :],
                dst_ref=rs_buf.at[src],
                send_sem=rs_send_sems.at[k],
                recv_sem=rs_recv_sems.at[k],
                device_id=(src,),
                device_id_type=pl.DeviceIdType.MESH,
            )
            rdma.wait_recv()
            own = own + rs_buf[src].astype(jnp.float32)

        own = _gelu(own)
        out_ref[pl.ds(me * CHUNK, CHUNK), :] = own
        ag_buf[me] = own.astype(jnp.bfloat16)

        ag_rdmas = []
        for k in range(1, N_DEV):
            peer = lax.rem(me + k, N_DEV)
            rdma = pltpu.make_async_remote_copy(
                src_ref=ag_buf.at[me],
                dst_ref=ag_buf.at[me],
                send_sem=ag_send_sems.at[k],
                recv_sem=ag_recv_sems.at[k],
                device_id=(peer,),
                device_id_type=pl.DeviceIdType.MESH,
            )
            rdma.start()
            ag_rdmas.append(rdma)

        for rdma in rs_rdmas:
            rdma.wait_send()

        for k in range(1, N_DEV):
            src = lax.rem(me + N_DEV - k, N_DEV)
            rdma = pltpu.make_async_remote_copy(
                src_ref=ag_buf.at[src],
                dst_ref=ag_buf.at[src],
                send_sem=ag_send_sems.at[k],
                recv_sem=ag_recv_sems.at[k],
                device_id=(src,),
                device_id_type=pl.DeviceIdType.MESH,
            )
            rdma.wait_recv()
            out_ref[pl.ds(src * CHUNK, CHUNK), :] = (
                ag_buf[src].astype(jnp.float32)
            )
        for rdma in ag_rdmas:
            rdma.wait_send()

        @functools.partial(
            pl.run_scoped, second_barrier=pltpu.SemaphoreType.REGULAR
        )
        def _(second_barrier):
            for k in range(1, N_DEV):
                pl.semaphore_signal(
                    second_barrier, inc=1,
                    device_id=(lax.rem(me + k, N_DEV),),
                    device_id_type=pl.DeviceIdType.MESH,
                )
            pl.semaphore_wait(second_barrier, N_DEV - 1)

    return pl.pallas_call(
        body,
        out_shape=jax.ShapeDtypeStruct((M, N), jnp.float32),
        in_specs=[
            pl.BlockSpec(memory_space=pltpu.VMEM),
            pl.BlockSpec(memory_space=pltpu.VMEM),
        ],
        out_specs=pl.BlockSpec(memory_space=pltpu.VMEM),
        scratch_shapes=[
            pltpu.VMEM((M, N), jnp.float32),
            pltpu.VMEM((M, N), jnp.bfloat16),
            pltpu.VMEM((N_DEV, CHUNK, N), jnp.bfloat16),
            pltpu.VMEM((N_DEV, CHUNK, N), jnp.bfloat16),
            pltpu.SemaphoreType.DMA((N_DEV,)),
            pltpu.SemaphoreType.DMA((N_DEV,)),
            pltpu.SemaphoreType.DMA((N_DEV,)),
            pltpu.SemaphoreType.DMA((N_DEV,)),
        ],
        compiler_params=pltpu.CompilerParams(collective_id=0),
    )(x, w_mat)


# device time: 64074 ns/iter; 3.2990x vs baseline; 1.0507x over previous
import functools

import jax
import jax.numpy as jnp
from jax import lax
from jax.experimental import pallas as pl
from jax.experimental.pallas import tpu as pltpu

N_DEV = 32
M = 1024
N = 1024
CHUNK = M // N_DEV
HALF = CHUNK // 2


def _gelu(y):
    c = 0.7978845608028654
    return 0.5 * y * (1.0 + jnp.tanh(c * (y + 0.044715 * y * y * y)))


def kernel(x, w_mat):
    def body(x_ref, w_ref, out_ref, acc_ref, acc16_ref, rs_buf, ag_buf,
             rs_send_sems, rs_recv_sems, ag_send_sems, ag_recv_sems):
        me = lax.axis_index("i")

        barrier_sem = pltpu.get_barrier_semaphore()
        for k in range(1, N_DEV):
            pl.semaphore_signal(
                barrier_sem, inc=1,
                device_id=(lax.rem(me + k, N_DEV),),
                device_id_type=pl.DeviceIdType.MESH,
            )
        pl.semaphore_wait(barrier_sem, N_DEV - 1)

        acc_ref[...] = jnp.dot(
            x_ref[...], w_ref[...], preferred_element_type=jnp.float32
        )
        acc16_ref[...] = acc_ref[...].astype(jnp.bfloat16)

        rs_rdmas = []
        for h in range(2):
            for k in range(1, N_DEV):
                peer = lax.rem(me + k, N_DEV)
                rdma = pltpu.make_async_remote_copy(
                    src_ref=acc16_ref.at[
                        pl.ds(peer * CHUNK + h * HALF, HALF), :
                    ],
                    dst_ref=rs_buf.at[h, me],
                    send_sem=rs_send_sems.at[h, k],
                    recv_sem=rs_recv_sems.at[h, k],
                    device_id=(peer,),
                    device_id_type=pl.DeviceIdType.MESH,
                )
                rdma.start()
                rs_rdmas.append(rdma)

        ag_rdmas = []

        def rs_wait_accumulate_broadcast(h):
            own = acc_ref[pl.ds(me * CHUNK + h * HALF, HALF), :]
            for k in range(1, N_DEV):
                src = lax.rem(me + N_DEV - k, N_DEV)
                rdma = pltpu.make_async_remote_copy(
                    src_ref=acc16_ref.at[pl.ds(0, HALF), :],
                    dst_ref=rs_buf.at[h, src],
                    send_sem=rs_send_sems.at[h, k],
                    recv_sem=rs_recv_sems.at[h, k],
                    device_id=(src,),
                    device_id_type=pl.DeviceIdType.MESH,
                )
                rdma.wait_recv()
                own = own + rs_buf[h, src].astype(jnp.float32)

            own = _gelu(own)
            out_ref[pl.ds(me * CHUNK + h * HALF, HALF), :] = own
            ag_buf[h, me] = own.astype(jnp.bfloat16)

            for k in range(1, N_DEV):
                peer = lax.rem(me + k, N_DEV)
                rdma = pltpu.make_async_remote_copy(
                    src_ref=ag_buf.at[h, me],
                    dst_ref=ag_buf.at[h, me],
                    send_sem=ag_send_sems.at[h, k],
                    recv_sem=ag_recv_sems.at[h, k],
                    device_id=(peer,),
                    device_id_type=pl.DeviceIdType.MESH,
                )
                rdma.start()
                ag_rdmas.append(rdma)

        def ag_wait_store(h):
            for k in range(1, N_DEV):
                src = lax.rem(me + N_DEV - k, N_DEV)
                rdma = pltpu.make_async_remote_copy(
                    src_ref=ag_buf.at[h, src],
                    dst_ref=ag_buf.at[h, src],
                    send_sem=ag_send_sems.at[h, k],
                    recv_sem=ag_recv_sems.at[h, k],
                    device_id=(src,),
                    device_id_type=pl.DeviceIdType.MESH,
                )
                rdma.wait_recv()
                out_ref[pl.ds(src * CHUNK + h * HALF, HALF), :] = (
                    ag_buf[h, src].astype(jnp.float32)
                )

        rs_wait_accumulate_broadcast(0)
        rs_wait_accumulate_broadcast(1)
        for rdma in rs_rdmas:
            rdma.wait_send()
        ag_wait_store(0)
        ag_wait_store(1)
        for rdma in ag_rdmas:
            rdma.wait_send()

        @functools.partial(
            pl.run_scoped, second_barrier=pltpu.SemaphoreType.REGULAR
        )
        def _(second_barrier):
            for k in range(1, N_DEV):
                pl.semaphore_signal(
                    second_barrier, inc=1,
                    device_id=(lax.rem(me + k, N_DEV),),
                    device_id_type=pl.DeviceIdType.MESH,
                )
            pl.semaphore_wait(second_barrier, N_DEV - 1)

    return pl.pallas_call(
        body,
        out_shape=jax.ShapeDtypeStruct((M, N), jnp.float32),
        in_specs=[
            pl.BlockSpec(memory_space=pltpu.VMEM),
            pl.BlockSpec(memory_space=pltpu.VMEM),
        ],
        out_specs=pl.BlockSpec(memory_space=pltpu.VMEM),
        scratch_shapes=[
            pltpu.VMEM((M, N), jnp.float32),
            pltpu.VMEM((M, N), jnp.bfloat16),
            pltpu.VMEM((2, N_DEV, HALF, N), jnp.bfloat16),
            pltpu.VMEM((2, N_DEV, HALF, N), jnp.bfloat16),
            pltpu.SemaphoreType.DMA((2, N_DEV)),
            pltpu.SemaphoreType.DMA((2, N_DEV)),
            pltpu.SemaphoreType.DMA((2, N_DEV)),
            pltpu.SemaphoreType.DMA((2, N_DEV)),
        ],
        compiler_params=pltpu.CompilerParams(collective_id=0),
    )(x, w_mat)


# device time: 61871 ns/iter; 3.4165x vs baseline; 1.0356x over previous
import functools
import os

import jax
import jax.numpy as jnp
from jax import lax
from jax.experimental import pallas as pl
from jax.experimental.pallas import tpu as pltpu

_VARIANT = os.environ.get("KERNEL_VARIANT", "full")

N_DEV = 32
M = 1024
N = 1024
CHUNK = M // N_DEV
HALF = CHUNK // 2


def _gelu(y):
    c = 0.7978845608028654
    return 0.5 * y * (1.0 + jnp.tanh(c * (y + 0.044715 * y * y * y)))


def kernel(x, w_mat):
    def body(x_ref, w_ref, out_ref, acc_ref, acc16_ref, rs_buf, ag_buf,
             rs_send_sems, rs_recv_sems, ag_send_sems, ag_recv_sems):
        me = lax.axis_index("i")

        barrier_sem = pltpu.get_barrier_semaphore()
        for k in range(1, N_DEV):
            pl.semaphore_signal(
                barrier_sem, inc=1,
                device_id=(lax.rem(me + k, N_DEV),),
                device_id_type=pl.DeviceIdType.MESH,
            )
        pl.semaphore_wait(barrier_sem, N_DEV - 1)

        acc_ref[...] = jnp.dot(
            x_ref[...], w_ref[...], preferred_element_type=jnp.float32
        )
        acc16_ref[...] = acc_ref[...].astype(jnp.bfloat16)

        rs_rdmas = []
        for h in range(2) if _VARIANT != "local_only" else []:
            for k in range(1, N_DEV):
                peer = lax.rem(me + k, N_DEV)
                rdma = pltpu.make_async_remote_copy(
                    src_ref=acc16_ref.at[
                        pl.ds(peer * CHUNK + h * HALF, HALF), :
                    ],
                    dst_ref=rs_buf.at[h, me],
                    send_sem=rs_send_sems.at[h, k],
                    recv_sem=rs_recv_sems.at[h, k],
                    device_id=(peer,),
                    device_id_type=pl.DeviceIdType.MESH,
                )
                rdma.start()
                rs_rdmas.append(rdma)

        ag_rdmas = []

        def rs_wait_accumulate_broadcast(h):
            rs_buf[h, me] = acc16_ref[pl.ds(me * CHUNK + h * HALF, HALF), :]
            for k in range(1, N_DEV) if _VARIANT != "local_only" else []:
                src = lax.rem(me + N_DEV - k, N_DEV)
                rdma = pltpu.make_async_remote_copy(
                    src_ref=acc16_ref.at[pl.ds(0, HALF), :],
                    dst_ref=rs_buf.at[h, src],
                    send_sem=rs_send_sems.at[h, k],
                    recv_sem=rs_recv_sems.at[h, k],
                    device_id=(src,),
                    device_id_type=pl.DeviceIdType.MESH,
                )
                rdma.wait_recv()

            own = jnp.sum(rs_buf[h].astype(jnp.float32), axis=0)
            own = _gelu(own)
            out_ref[pl.ds(me * CHUNK + h * HALF, HALF), :] = own
            ag_buf[h, me] = own.astype(jnp.bfloat16)

            for k in range(1, N_DEV) if _VARIANT == "full" else []:
                peer = lax.rem(me + k, N_DEV)
                rdma = pltpu.make_async_remote_copy(
                    src_ref=ag_buf.at[h, me],
                    dst_ref=ag_buf.at[h, me],
                    send_sem=ag_send_sems.at[h, k],
                    recv_sem=ag_recv_sems.at[h, k],
                    device_id=(peer,),
                    device_id_type=pl.DeviceIdType.MESH,
                )
                rdma.start()
                ag_rdmas.append(rdma)

        def ag_wait_store(h):
            for k in range(1, N_DEV) if _VARIANT == "full" else []:
                src = lax.rem(me + N_DEV - k, N_DEV)
                rdma = pltpu.make_async_remote_copy(
                    src_ref=ag_buf.at[h, src],
                    dst_ref=ag_buf.at[h, src],
                    send_sem=ag_send_sems.at[h, k],
                    recv_sem=ag_recv_sems.at[h, k],
                    device_id=(src,),
                    device_id_type=pl.DeviceIdType.MESH,
                )
                rdma.wait_recv()
                out_ref[pl.ds(src * CHUNK + h * HALF, HALF), :] = (
                    ag_buf[h, src].astype(jnp.float32)
                )

        rs_wait_accumulate_broadcast(0)
        rs_wait_accumulate_broadcast(1)
        for rdma in rs_rdmas:
            rdma.wait_send()
        ag_wait_store(0)
        ag_wait_store(1)
        for rdma in ag_rdmas:
            rdma.wait_send()

    return pl.pallas_call(
        body,
        out_shape=jax.ShapeDtypeStruct((M, N), jnp.float32),
        in_specs=[
            pl.BlockSpec(memory_space=pltpu.VMEM),
            pl.BlockSpec(memory_space=pltpu.VMEM),
        ],
        out_specs=pl.BlockSpec(memory_space=pltpu.VMEM),
        scratch_shapes=[
            pltpu.VMEM((M, N), jnp.float32),
            pltpu.VMEM((M, N), jnp.bfloat16),
            pltpu.VMEM((2, N_DEV, HALF, N), jnp.bfloat16),
            pltpu.VMEM((2, N_DEV, HALF, N), jnp.bfloat16),
            pltpu.SemaphoreType.DMA((2, N_DEV)),
            pltpu.SemaphoreType.DMA((2, N_DEV)),
            pltpu.SemaphoreType.DMA((2, N_DEV)),
            pltpu.SemaphoreType.DMA((2, N_DEV)),
        ],
        compiler_params=pltpu.CompilerParams(collective_id=0),
    )(x, w_mat)


# device time: 57535 ns/iter; 3.6740x vs baseline; 1.0754x over previous
import os

import jax
import jax.numpy as jnp
from jax import lax
from jax.experimental import pallas as pl
from jax.experimental.pallas import tpu as pltpu

N_DEV = 32
M = 1024
N = 1024
CHUNK = M // N_DEV
HN = N // 2

_VARIANT = os.environ.get("KERNEL_VARIANT", "full")


def _gelu(y):
    c = 0.7978845608028654
    return 0.5 * y * (1.0 + jnp.tanh(c * (y + 0.044715 * y * y * y)))


def kernel(x, w_mat):
    def body(x_ref, w_ref, out_ref, acc16_ref, out16_ref, rs_buf,
             rs_send_sems, rs_recv_sems, ag_send_sems, ag_recv_sems):
        me = lax.axis_index("i")

        barrier_sem = pltpu.get_barrier_semaphore()
        for k in range(1, N_DEV):
            pl.semaphore_signal(
                barrier_sem, inc=1,
                device_id=(lax.rem(me + k, N_DEV),),
                device_id_type=pl.DeviceIdType.MESH,
            )

        acc16_ref[...] = jnp.dot(
            x_ref[...], w_ref[...], preferred_element_type=jnp.float32
        ).astype(jnp.bfloat16)
        for h in range(2):
            rs_buf[h, me] = acc16_ref[
                pl.ds(me * CHUNK, CHUNK), pl.ds(h * HN, HN)
            ]

        pl.semaphore_wait(barrier_sem, N_DEV - 1)

        rs_rdmas = []
        for h in range(2) if _VARIANT != "local_only" else []:
            for k in range(1, N_DEV):
                peer = lax.rem(me + k, N_DEV)
                rdma = pltpu.make_async_remote_copy(
                    src_ref=acc16_ref.at[
                        pl.ds(peer * CHUNK, CHUNK), pl.ds(h * HN, HN)
                    ],
                    dst_ref=rs_buf.at[h, me],
                    send_sem=rs_send_sems.at[h, k],
                    recv_sem=rs_recv_sems.at[h, k],
                    device_id=(peer,),
                    device_id_type=pl.DeviceIdType.MESH,
                )
                rdma.start()
                rs_rdmas.append(rdma)

        ag_rdmas = []

        def rs_wait_accumulate_broadcast(h):
            for k in range(1, N_DEV) if _VARIANT != "local_only" else []:
                src = lax.rem(me + N_DEV - k, N_DEV)
                rdma = pltpu.make_async_remote_copy(
                    src_ref=rs_buf.at[h, me],
                    dst_ref=rs_buf.at[h, src],
                    send_sem=rs_send_sems.at[h, k],
                    recv_sem=rs_recv_sems.at[h, k],
                    device_id=(src,),
                    device_id_type=pl.DeviceIdType.MESH,
                )
                rdma.wait_recv()

            own = jnp.sum(rs_buf[h].astype(jnp.float32), axis=0)
            out16_ref[pl.ds(me * CHUNK, CHUNK), pl.ds(h * HN, HN)] = (
                _gelu(own).astype(jnp.bfloat16)
            )

            for k in range(1, N_DEV) if _VARIANT == "full" else []:
                peer = lax.rem(me + k, N_DEV)
                rdma = pltpu.make_async_remote_copy(
                    src_ref=out16_ref.at[
                        pl.ds(me * CHUNK, CHUNK), pl.ds(h * HN, HN)
                    ],
                    dst_ref=out16_ref.at[
                        pl.ds(me * CHUNK, CHUNK), pl.ds(h * HN, HN)
                    ],
                    send_sem=ag_send_sems.at[h, k],
                    recv_sem=ag_recv_sems.at[h, k],
                    device_id=(peer,),
                    device_id_type=pl.DeviceIdType.MESH,
                )
                rdma.start()
                ag_rdmas.append(rdma)

        def ag_wait_widen(h):
            for k in range(1, N_DEV) if _VARIANT == "full" else []:
                src = lax.rem(me + N_DEV - k, N_DEV)
                rdma = pltpu.make_async_remote_copy(
                    src_ref=out16_ref.at[
                        pl.ds(me * CHUNK, CHUNK), pl.ds(h * HN, HN)
                    ],
                    dst_ref=out16_ref.at[
                        pl.ds(src * CHUNK, CHUNK), pl.ds(h * HN, HN)
                    ],
                    send_sem=ag_send_sems.at[h, k],
                    recv_sem=ag_recv_sems.at[h, k],
                    device_id=(src,),
                    device_id_type=pl.DeviceIdType.MESH,
                )
                rdma.wait_recv()
            out_ref[:, pl.ds(h * HN, HN)] = (
                out16_ref[:, pl.ds(h * HN, HN)].astype(jnp.float32)
            )

        rs_wait_accumulate_broadcast(0)
        rs_wait_accumulate_broadcast(1)
        for rdma in rs_rdmas:
            rdma.wait_send()
        ag_wait_widen(0)
        ag_wait_widen(1)
        for rdma in ag_rdmas:
            rdma.wait_send()

    return pl.pallas_call(
        body,
        out_shape=jax.ShapeDtypeStruct((M, N), jnp.float32),
        in_specs=[
            pl.BlockSpec(memory_space=pltpu.VMEM),
            pl.BlockSpec(memory_space=pltpu.VMEM),
        ],
        out_specs=pl.BlockSpec(memory_space=pltpu.VMEM),
        scratch_shapes=[
            pltpu.VMEM((M, N), jnp.bfloat16),
            pltpu.VMEM((M, N), jnp.bfloat16),
            pltpu.VMEM((2, N_DEV, CHUNK, HN), jnp.bfloat16),
            pltpu.SemaphoreType.DMA((2, N_DEV)),
            pltpu.SemaphoreType.DMA((2, N_DEV)),
            pltpu.SemaphoreType.DMA((2, N_DEV)),
            pltpu.SemaphoreType.DMA((2, N_DEV)),
        ],
        compiler_params=pltpu.CompilerParams(collective_id=0),
    )(x, w_mat)


# device time: 57273 ns/iter; 3.6908x vs baseline; 1.0046x over previous
import os

import jax
import jax.numpy as jnp
from jax import lax
from jax.experimental import pallas as pl
from jax.experimental.pallas import tpu as pltpu

N_DEV = 32
M = 1024
N = 1024
CHUNK = M // N_DEV
SPLITS = int(os.environ.get("KERNEL_SPLITS", "2"))
HN = N // SPLITS

_VARIANT = os.environ.get("KERNEL_VARIANT", "full")


def _gelu(y):
    c = 0.7978845608028654
    return 0.5 * y * (1.0 + jnp.tanh(c * (y + 0.044715 * y * y * y)))


def kernel(x, w_mat):
    def body(x_ref, w_ref, out_ref, acc16_ref, out16_ref, rs_buf,
             rs_send_sems, rs_recv_sems, ag_send_sems, ag_recv_sems):
        me = lax.axis_index("i")

        barrier_sem = pltpu.get_barrier_semaphore()
        for k in range(1, N_DEV):
            pl.semaphore_signal(
                barrier_sem, inc=1,
                device_id=(lax.rem(me + k, N_DEV),),
                device_id_type=pl.DeviceIdType.MESH,
            )

        xb = x_ref[...].astype(jnp.bfloat16)
        wb = w_ref[...].astype(jnp.bfloat16)
        rs_rdmas = []
        for h in range(SPLITS):
            acc16_ref[:, pl.ds(h * HN, HN)] = jnp.dot(
                xb, wb[:, h * HN:(h + 1) * HN],
                preferred_element_type=jnp.float32,
            ).astype(jnp.bfloat16)
            rs_buf[h, me] = acc16_ref[
                pl.ds(me * CHUNK, CHUNK), pl.ds(h * HN, HN)
            ]
            if h == 0:
                pl.semaphore_wait(barrier_sem, N_DEV - 1)
            if _VARIANT == "local_only":
                continue
            for k in range(1, N_DEV):
                peer = lax.rem(me + k, N_DEV)
                rdma = pltpu.make_async_remote_copy(
                    src_ref=acc16_ref.at[
                        pl.ds(peer * CHUNK, CHUNK), pl.ds(h * HN, HN)
                    ],
                    dst_ref=rs_buf.at[h, me],
                    send_sem=rs_send_sems.at[h, k],
                    recv_sem=rs_recv_sems.at[h, k],
                    device_id=(peer,),
                    device_id_type=pl.DeviceIdType.MESH,
                )
                rdma.start()
                rs_rdmas.append(rdma)

        ag_rdmas = []

        def rs_wait_accumulate_broadcast(h):
            for k in range(1, N_DEV) if _VARIANT != "local_only" else []:
                src = lax.rem(me + N_DEV - k, N_DEV)
                rdma = pltpu.make_async_remote_copy(
                    src_ref=rs_buf.at[h, me],
                    dst_ref=rs_buf.at[h, src],
                    send_sem=rs_send_sems.at[h, k],
                    recv_sem=rs_recv_sems.at[h, k],
                    device_id=(src,),
                    device_id_type=pl.DeviceIdType.MESH,
                )
                rdma.wait_recv()

            own = jnp.sum(rs_buf[h].astype(jnp.float32), axis=0)
            out16_ref[pl.ds(me * CHUNK, CHUNK), pl.ds(h * HN, HN)] = (
                _gelu(own).astype(jnp.bfloat16)
            )

            for k in range(1, N_DEV) if _VARIANT == "full" else []:
                peer = lax.rem(me + k, N_DEV)
                rdma = pltpu.make_async_remote_copy(
                    src_ref=out16_ref.at[
                        pl.ds(me * CHUNK, CHUNK), pl.ds(h * HN, HN)
                    ],
                    dst_ref=out16_ref.at[
                        pl.ds(me * CHUNK, CHUNK), pl.ds(h * HN, HN)
                    ],
                    send_sem=ag_send_sems.at[h, k],
                    recv_sem=ag_recv_sems.at[h, k],
                    device_id=(peer,),
                    device_id_type=pl.DeviceIdType.MESH,
                )
                rdma.start()
                ag_rdmas.append(rdma)

        def ag_wait_widen(h):
            for k in range(1, N_DEV) if _VARIANT == "full" else []:
                src = lax.rem(me + N_DEV - k, N_DEV)
                rdma = pltpu.make_async_remote_copy(
                    src_ref=out16_ref.at[
                        pl.ds(me * CHUNK, CHUNK), pl.ds(h * HN, HN)
                    ],
                    dst_ref=out16_ref.at[
                        pl.ds(src * CHUNK, CHUNK), pl.ds(h * HN, HN)
                    ],
                    send_sem=ag_send_sems.at[h, k],
                    recv_sem=ag_recv_sems.at[h, k],
                    device_id=(src,),
                    device_id_type=pl.DeviceIdType.MESH,
                )
                rdma.wait_recv()
            out_ref[:, pl.ds(h * HN, HN)] = (
                out16_ref[:, pl.ds(h * HN, HN)].astype(jnp.float32)
            )

        for h in range(SPLITS):
            rs_wait_accumulate_broadcast(h)
        for rdma in rs_rdmas:
            rdma.wait_send()
        for h in range(SPLITS):
            ag_wait_widen(h)
        for rdma in ag_rdmas:
            rdma.wait_send()

    return pl.pallas_call(
        body,
        out_shape=jax.ShapeDtypeStruct((M, N), jnp.float32),
        in_specs=[
            pl.BlockSpec(memory_space=pltpu.VMEM),
            pl.BlockSpec(memory_space=pltpu.VMEM),
        ],
        out_specs=pl.BlockSpec(memory_space=pltpu.VMEM),
        scratch_shapes=[
            pltpu.VMEM((M, N), jnp.bfloat16),
            pltpu.VMEM((M, N), jnp.bfloat16),
            pltpu.VMEM((SPLITS, N_DEV, CHUNK, HN), jnp.bfloat16),
            pltpu.SemaphoreType.DMA((SPLITS, N_DEV)),
            pltpu.SemaphoreType.DMA((SPLITS, N_DEV)),
            pltpu.SemaphoreType.DMA((SPLITS, N_DEV)),
            pltpu.SemaphoreType.DMA((SPLITS, N_DEV)),
        ],
        compiler_params=pltpu.CompilerParams(collective_id=0),
    )(x, w_mat)
